# baseline (speedup 1.0000x reference)
import jax
import jax.numpy as jnp
from jax import lax
from jax.experimental import pallas as pl
from jax.experimental.pallas import tpu as pltpu
from jax.experimental.pallas import tpu_sc as plsc
import functools

N = 320000
D = 128
S = 10000

NC = 2
NS = 16
DC = D // NC
ROWS_PER_TILE = N // NS
SEG_PER_TILE = S // NS
SUB = 80
NSUB = 2
BLK = SUB * NSUB
NBUF = 3
NBLK = ROWS_PER_TILE // BLK
W = 896
WSUB = 112
NWSUB = W // WSUB
LANES = 16


def _make_kernel():
    mesh = plsc.VectorSubcoreMesh(core_axis_name="c", subcore_axis_name="s")

    @functools.partial(
        pl.kernel,
        out_type=jax.ShapeDtypeStruct((S, D), jnp.float32),
        mesh=mesh,
        scratch_types=[
            pltpu.VMEM((NBUF, NSUB, SUB), jnp.int32),
            pltpu.VMEM((NBUF, BLK, DC), jnp.float32),
            pltpu.VMEM((W, DC), jnp.float32),
            pltpu.VMEM((NWSUB, WSUB), jnp.int32),
            pltpu.VMEM_SHARED((S, DC), jnp.float32),
            pltpu.SemaphoreType.DMA((NBUF,)),
            pltpu.SemaphoreType.DMA((NBUF,)),
            pltpu.SemaphoreType.DMA,
        ],
        compiler_params=pltpu.CompilerParams(
            use_tc_tiling_on_sc=False, needs_layout_passes=False),
    )
    def seg_sum(data_hbm, seg_hbm, zeros_hbm, out_hbm,
                idx_v, rows_v, acc_loc, idx_w, acc_sh, sem_i, sem_d, sem_s):
        c = lax.axis_index("c")
        s = lax.axis_index("s")
        row0 = s * ROWS_PER_TILE
        col0 = c * DC

        def start_load(g, b):
            r = row0 + g * BLK
            pltpu.async_copy(
                seg_hbm.at[pl.ds(r // SUB, NSUB)], idx_v.at[b], sem_i.at[b])
            pltpu.async_copy(
                data_hbm.at[pl.ds(r, BLK), pl.ds(col0, DC)], rows_v.at[b],
                sem_d.at[b])

        def wait_load(b):
            pltpu.make_async_copy(
                seg_hbm.at[pl.ds(0, NSUB)], idx_v.at[b], sem_i.at[b]).wait()
            pltpu.make_async_copy(
                data_hbm.at[pl.ds(0, BLK), pl.ds(col0, DC)], rows_v.at[b],
                sem_d.at[b]).wait()

        start_load(0, 0)
        start_load(1, 1)
        pltpu.sync_copy(zeros_hbm, acc_loc)
        pltpu.sync_copy(zeros_hbm.at[pl.ds(0, SEG_PER_TILE)],
                        acc_sh.at[pl.ds(s * SEG_PER_TILE, SEG_PER_TILE)])
        plsc.subcore_barrier()

        def body(g, lo):
            b = lax.rem(g, NBUF)

            @pl.when(g + 2 < NBLK)
            def _():
                start_load(g + 2, lax.rem(g + 2, NBUF))

            wait_load(b)
            first16 = idx_v[b, 0, pl.ds(0, LANES)]
            lo = lax.select(g == 0, first16[0], lo)
            last16 = idx_v[b, NSUB - 1, pl.ds(SUB - LANES, LANES)]
            in_window = last16[LANES - 1] - lo < W

            @pl.when(in_window)
            def _():
                iota16 = lax.iota(jnp.int32, LANES)
                for k in range(BLK // LANES):
                    ids16 = idx_v[b, k // (SUB // LANES),
                                  pl.ds((k % (SUB // LANES)) * LANES, LANES)]
                    w16 = ids16 - lo
                    rows16 = k * LANES + iota16
                    zero16 = iota16 * 0
                    for ccol in range(DC):
                        cv = zero16 + ccol
                        val = plsc.load_gather(rows_v.at[b], [rows16, cv])
                        plsc.addupdate_scatter(acc_loc, [w16, cv], val)

            @pl.when(jnp.logical_not(in_window))
            def _():
                descs = [
                    pltpu.async_copy(
                        rows_v.at[b, pl.ds(j * SUB, SUB)],
                        acc_sh.at[idx_v.at[b, j]], sem_s, add=True)
                    for j in range(NSUB)
                ]
                for d in descs:
                    d.wait()

            return lo

        lo = lax.fori_loop(0, NBLK, body, 0)

        iota = lax.iota(jnp.int32, LANES)
        for jw in range(NWSUB):
            for k in range(WSUB // LANES):
                idx_w[jw, pl.ds(k * LANES, LANES)] = jnp.minimum(
                    lo + jw * WSUB + k * LANES + iota, S - 1)
        descs = [
            pltpu.async_copy(
                acc_loc.at[pl.ds(jw * WSUB, WSUB)],
                acc_sh.at[idx_w.at[jw]], sem_s, add=True)
            for jw in range(NWSUB)
        ]
        for d in descs:
            d.wait()
        plsc.subcore_barrier()

        pltpu.sync_copy(
            acc_sh.at[pl.ds(s * SEG_PER_TILE, SEG_PER_TILE)],
            out_hbm.at[pl.ds(s * SEG_PER_TILE, SEG_PER_TILE), pl.ds(col0, DC)],
        )

    return seg_sum


_seg_sum = _make_kernel()


def kernel(data, segment_ids):
    ids = segment_ids.astype(jnp.int32).reshape(N // SUB, SUB)
    zeros = jnp.zeros((W, DC), jnp.float32)
    return _seg_sum(data, ids, zeros)


# speedup vs baseline: 16.4707x; 16.4707x over previous
import jax
import jax.numpy as jnp
from jax import lax
from jax.experimental import pallas as pl
from jax.experimental.pallas import tpu as pltpu
from jax.experimental.pallas import tpu_sc as plsc
import functools

N = 320000
D = 128
S = 10000

NC = 2
NS = 16
DC = D // NC
ROWS_PER_TILE = N // NS
SEG_PER_TILE = S // NS
SUB = 100
NSUB = 4
BLK = SUB * NSUB
NBUF = 3
NBLK = ROWS_PER_TILE // BLK


def _make_kernel():
    mesh = plsc.VectorSubcoreMesh(core_axis_name="c", subcore_axis_name="s")

    @functools.partial(
        pl.kernel,
        out_type=jax.ShapeDtypeStruct((S, D), jnp.float32),
        mesh=mesh,
        scratch_types=[
            pltpu.VMEM((NBUF, NSUB, SUB), jnp.int32),
            pltpu.VMEM((NBUF, BLK, DC), jnp.float32),
            pltpu.VMEM_SHARED((S, DC), jnp.float32),
            pltpu.SemaphoreType.DMA((NBUF,)),
            pltpu.SemaphoreType.DMA((NBUF,)),
            pltpu.SemaphoreType.DMA,
        ],
        compiler_params=pltpu.CompilerParams(use_tc_tiling_on_sc=False),
    )
    def seg_sum(data_hbm, seg_hbm, zeros_hbm, out_hbm,
                idx_v, rows_v, acc_sh, sem_i, sem_d, sem_s):
        c = lax.axis_index("c")
        s = lax.axis_index("s")
        row0 = s * ROWS_PER_TILE
        col0 = c * DC

        def start_load(g, b):
            r = row0 + g * BLK
            pltpu.async_copy(
                seg_hbm.at[pl.ds(r // SUB, NSUB)], idx_v.at[b], sem_i.at[b])
            pltpu.async_copy(
                data_hbm.at[pl.ds(r, BLK), pl.ds(col0, DC)], rows_v.at[b],
                sem_d.at[b])

        def wait_load(b):
            pltpu.make_async_copy(
                seg_hbm.at[pl.ds(0, NSUB)], idx_v.at[b], sem_i.at[b]).wait()
            pltpu.make_async_copy(
                data_hbm.at[pl.ds(0, BLK), pl.ds(col0, DC)], rows_v.at[b],
                sem_d.at[b]).wait()

        start_load(0, 0)
        start_load(1, 1)
        pltpu.sync_copy(zeros_hbm, acc_sh.at[pl.ds(s * SEG_PER_TILE, SEG_PER_TILE)])
        plsc.subcore_barrier()

        def body(g, carry):
            b = lax.rem(g, NBUF)

            @pl.when(g + 2 < NBLK)
            def _():
                start_load(g + 2, lax.rem(g + 2, NBUF))

            wait_load(b)
            descs = [
                pltpu.async_copy(
                    rows_v.at[b, pl.ds(j * SUB, SUB)],
                    acc_sh.at[idx_v.at[b, j]], sem_s, add=True)
                for j in range(NSUB)
            ]
            for d in descs:
                d.wait()
            return carry

        lax.fori_loop(0, NBLK, body, 0)
        plsc.subcore_barrier()

        pltpu.sync_copy(
            acc_sh.at[pl.ds(s * SEG_PER_TILE, SEG_PER_TILE)],
            out_hbm.at[pl.ds(s * SEG_PER_TILE, SEG_PER_TILE), pl.ds(col0, DC)],
        )

    return seg_sum


_seg_sum = _make_kernel()


def kernel(data, segment_ids):
    ids = segment_ids.astype(jnp.int32).reshape(N // SUB, SUB)
    zeros = jnp.zeros((SEG_PER_TILE, DC), jnp.float32)
    return _seg_sum(data, ids, zeros)
